# baseline (speedup 1.0000x reference)
import functools

import jax
import jax.numpy as jnp
from jax import lax
from jax.experimental import pallas as pl
from jax.experimental.pallas import tpu as pltpu
from jax.experimental.pallas import tpu_sc as plsc

S = 2048
D = 768
H = 12
DH = D // H
VP = 2 * DH
NW = 32
RW = H * S // NW
NCH = RW // 128
CPH = S // 128

_DN_T = (((1,), (1,)), ((), ()))
_DN_N = (((1,), (0,)), ((), ()))


def _score_idx_body(x_ref, wq_ref, bq_ref, wk_ref, bk_ref, wv_ref, bv_ref,
                    v_ref, idx_ref, q_scr, k_scr):
    h = pl.program_id(0)

    @pl.when(h == 0)
    def _():
        xb = x_ref[...]
        qb = jax.lax.dot_general(
            xb, wq_ref[...], _DN_T, preferred_element_type=jnp.float32) + bq_ref[...]
        kb = jax.lax.dot_general(
            xb, wk_ref[...], _DN_T, preferred_element_type=jnp.float32) + bk_ref[...]
        vb = jax.lax.dot_general(
            xb, wv_ref[...], _DN_T, preferred_element_type=jnp.float32) + bv_ref[...]
        for hh in range(H):
            sl = slice(hh * DH, (hh + 1) * DH)
            q_scr[hh] = qb[:, sl]
            k_scr[hh] = kb[:, sl]
            v_ref[hh, :, 0:DH] = vb[:, sl]
            v_ref[hh, :, DH:VP] = vb[:, sl]

    s = jax.lax.dot_general(q_scr[h], k_scr[h], _DN_T,
                            preferred_element_type=jnp.float32)
    idx = jnp.argmax(s, axis=1).astype(jnp.int32) + h * S
    idx_ref[...] = idx.reshape(CPH, 128)


def _make_gather():
    mesh = plsc.VectorSubcoreMesh(core_axis_name="c", subcore_axis_name="s")

    @functools.partial(
        pl.kernel, mesh=mesh,
        out_type=jax.ShapeDtypeStruct((H * S, VP), jnp.float32),
        scratch_types=[
            pltpu.VMEM((NCH, 128), jnp.int32),
            pltpu.VMEM((RW, VP), jnp.float32),
            pltpu.SemaphoreType.DMA,
        ],
    )
    def gather(table_hbm, idx_hbm, out_hbm, idx_v, rows_v, sem):
        wid = lax.axis_index("s") * 2 + lax.axis_index("c")
        pltpu.sync_copy(idx_hbm.at[wid], idx_v)
        copies = [pltpu.async_copy(table_hbm.at[idx_v.at[j]],
                                   rows_v.at[pl.ds(j * 128, 128)], sem)
                  for j in range(NCH)]
        for c in copies:
            c.wait()
        pltpu.sync_copy(rows_v, out_hbm.at[pl.ds(wid * RW, RW)])

    return gather


def _proj_body(att_ref, wot_ref, bo_ref, out_ref):
    h = pl.program_id(0)
    proj = jax.lax.dot_general(att_ref[0], wot_ref[0], _DN_N,
                               preferred_element_type=jnp.float32)

    @pl.when(h == 0)
    def _():
        out_ref[...] = proj + bo_ref[...]

    @pl.when(h != 0)
    def _():
        out_ref[...] += proj


def kernel(x, Wq, bq, Wk, bk, Wv, bv, Wo, bo):
    x2 = x.reshape(S, D)
    bq2 = bq.reshape(1, D)
    bk2 = bk.reshape(1, D)
    bv2 = bv.reshape(1, D)
    bo2 = bo.reshape(1, D)
    WoT = Wo.T.reshape(H, DH, D)
    WoTp = jnp.concatenate([WoT, jnp.zeros_like(WoT)], axis=1)

    w_spec = pl.BlockSpec((D, D), lambda h: (0, 0))
    b_spec = pl.BlockSpec((1, D), lambda h: (0, 0))
    v, idx = pl.pallas_call(
        _score_idx_body,
        grid=(H,),
        in_specs=[pl.BlockSpec((S, D), lambda h: (0, 0)),
                  w_spec, b_spec, w_spec, b_spec, w_spec, b_spec],
        out_specs=[pl.BlockSpec((H, S, VP), lambda h: (0, 0, 0)),
                   pl.BlockSpec((CPH, 128), lambda h: (h, 0))],
        out_shape=[jax.ShapeDtypeStruct((H, S, VP), jnp.float32),
                   jax.ShapeDtypeStruct((H * CPH, 128), jnp.int32)],
        scratch_shapes=[pltpu.VMEM((H, S, DH), jnp.float32),
                        pltpu.VMEM((H, S, DH), jnp.float32)],
    )(x2, Wq, bq2, Wk, bk2, Wv, bv2)

    att = _make_gather()(v.reshape(H * S, VP),
                         idx.reshape(NW, NCH, 128))

    out = pl.pallas_call(
        _proj_body,
        grid=(H,),
        in_specs=[
            pl.BlockSpec((1, S, VP), lambda h: (h, 0, 0)),
            pl.BlockSpec((1, VP, D), lambda h: (h, 0, 0)),
            pl.BlockSpec((1, D), lambda h: (0, 0)),
        ],
        out_specs=pl.BlockSpec((S, D), lambda h: (0, 0)),
        out_shape=jax.ShapeDtypeStruct((S, D), jnp.float32),
    )(att.reshape(H, S, VP), WoTp, bo2)

    return out.reshape(1, S, D)


# speedup vs baseline: 1.0207x; 1.0207x over previous
import functools

import jax
import jax.numpy as jnp
from jax import lax
from jax.experimental import pallas as pl
from jax.experimental.pallas import tpu as pltpu
from jax.experimental.pallas import tpu_sc as plsc

S = 2048
D = 768
H = 12
DH = D // H
VP = 2 * DH
NW = 32
RW = H * S // NW
NCH = RW // 128
CPH = S // 128

_DN_T = (((1,), (1,)), ((), ()))
_DN_N = (((1,), (0,)), ((), ()))


def _score_idx_body(x_ref, wq_ref, bq_ref, wk_ref, bk_ref, wv_ref, bv_ref,
                    v_ref, idx_ref, q_scr, k_scr):
    h = pl.program_id(0)

    @pl.when(h == 0)
    def _():
        xb = x_ref[...]
        qb = jax.lax.dot_general(
            xb, wq_ref[...], _DN_T, preferred_element_type=jnp.float32) + bq_ref[...]
        kb = jax.lax.dot_general(
            xb, wk_ref[...], _DN_T, preferred_element_type=jnp.float32) + bk_ref[...]
        vb = jax.lax.dot_general(
            xb, wv_ref[...], _DN_T, preferred_element_type=jnp.float32) + bv_ref[...]
        for hh in range(H):
            sl = slice(hh * DH, (hh + 1) * DH)
            q_scr[hh] = qb[:, sl]
            k_scr[hh] = kb[:, sl]
            v_ref[hh, :, 0:DH] = vb[:, sl]
            v_ref[hh, :, DH:VP] = vb[:, sl]

    s = jax.lax.dot_general(q_scr[h], k_scr[h], _DN_T,
                            preferred_element_type=jnp.float32)
    idx = jnp.argmax(s, axis=1).astype(jnp.int32) + h * S
    idx_ref[...] = idx.reshape(CPH, 128)


def _make_gather():
    mesh = plsc.VectorSubcoreMesh(core_axis_name="c", subcore_axis_name="s")

    @functools.partial(
        pl.kernel, mesh=mesh,
        out_type=jax.ShapeDtypeStruct((H * S, VP), jnp.float32),
        scratch_types=[
            pltpu.VMEM((16, 128), jnp.int32),
            pltpu.VMEM((RW, VP), jnp.float32),
            pltpu.SemaphoreType.DMA,
        ],
    )
    def gather(table_hbm, idx_hbm, out_hbm, idx_v, rows_v, sem):
        wid = lax.axis_index("s") * 2 + lax.axis_index("c")
        first = wid * NCH
        base = pl.multiple_of((first // 8) * 8, 8)
        off = first - base
        pltpu.sync_copy(idx_hbm.at[pl.ds(base, 16)], idx_v)
        copies = [pltpu.async_copy(table_hbm.at[idx_v.at[off + j]],
                                   rows_v.at[pl.ds(j * 128, 128)], sem)
                  for j in range(NCH)]
        for c in copies:
            c.wait()
        pltpu.sync_copy(rows_v, out_hbm.at[pl.ds(wid * RW, RW)])

    return gather


def _proj_body(att_ref, wot_ref, bo_ref, out_ref):
    h = pl.program_id(0)
    proj = jax.lax.dot_general(att_ref[0, :, 0:DH], wot_ref[...], _DN_N,
                               preferred_element_type=jnp.float32)

    @pl.when(h == 0)
    def _():
        out_ref[...] = proj + bo_ref[...]

    @pl.when(h != 0)
    def _():
        out_ref[...] += proj


def kernel(x, Wq, bq, Wk, bk, Wv, bv, Wo, bo):
    x2 = x.reshape(S, D)
    bq2 = bq.reshape(1, D)
    bk2 = bk.reshape(1, D)
    bv2 = bv.reshape(1, D)
    bo2 = bo.reshape(1, D)
    WoT = Wo.T

    w_spec = pl.BlockSpec((D, D), lambda h: (0, 0))
    b_spec = pl.BlockSpec((1, D), lambda h: (0, 0))
    v, idx = pl.pallas_call(
        _score_idx_body,
        grid=(H,),
        in_specs=[pl.BlockSpec((S, D), lambda h: (0, 0)),
                  w_spec, b_spec, w_spec, b_spec, w_spec, b_spec],
        out_specs=[pl.BlockSpec((H, S, VP), lambda h: (0, 0, 0)),
                   pl.BlockSpec((CPH, 128), lambda h: (h, 0))],
        out_shape=[jax.ShapeDtypeStruct((H, S, VP), jnp.float32),
                   jax.ShapeDtypeStruct((208, 128), jnp.int32)],
        scratch_shapes=[pltpu.VMEM((H, S, DH), jnp.float32),
                        pltpu.VMEM((H, S, DH), jnp.float32)],
    )(x2, Wq, bq2, Wk, bk2, Wv, bv2)

    att = _make_gather()(v.reshape(H * S, VP), idx)

    out = pl.pallas_call(
        _proj_body,
        grid=(H,),
        in_specs=[
            pl.BlockSpec((1, S, VP), lambda h: (h, 0, 0)),
            pl.BlockSpec((DH, D), lambda h: (h, 0)),
            pl.BlockSpec((1, D), lambda h: (0, 0)),
        ],
        out_specs=pl.BlockSpec((S, D), lambda h: (0, 0)),
        out_shape=jax.ShapeDtypeStruct((S, D), jnp.float32),
    )(att.reshape(H, S, VP), WoT, bo2)

    return out.reshape(1, S, D)
